# baseline (device time: 10789 ns/iter reference)
import jax
import jax.numpy as jnp
from jax import lax
from jax.experimental import pallas as pl
from jax.experimental.pallas import tpu as pltpu

M = 512
D = 512
C = 8
CH = M // C


def kernel(partial, gamma):
    def body(p_ref, g_ref, o_ref, peer_ref, loc_ref, ob_ref,
             txq_ref, rxq_ref, txs_ref, rxs_ref,
             fetch_sems, st_sems, sc_sems, send_sems, recv_sems):
        my_x = lax.axis_index("x")
        my_y = lax.axis_index("y")
        my_z = lax.axis_index("z")
        q = 1 - my_y
        y_peer = (my_x, q, my_z)

        cp_peer = pltpu.make_async_copy(
            p_ref.at[0].at[pl.ds(q * M, M), :], peer_ref, fetch_sems.at[0]
        )
        cp_loc = pltpu.make_async_copy(
            p_ref.at[0].at[pl.ds(my_y * M, M), :], loc_ref, fetch_sems.at[1]
        )
        cp_peer.start()
        cp_loc.start()

        barrier_sem = pltpu.get_barrier_semaphore()
        pl.semaphore_signal(
            barrier_sem, inc=1,
            device_id=y_peer, device_id_type=pl.DeviceIdType.MESH,
        )

        cp_peer.wait()
        b = peer_ref[:, :]
        s = jnp.max(jnp.abs(b), axis=0, keepdims=True)
        txs_ref[:, :] = s * (1.0 / 127.0)
        txq_ref[:, :] = jnp.round(b * (127.0 / jnp.maximum(s, 1e-30))).astype(jnp.int8)

        pl.semaphore_wait(barrier_sem, 1)

        sc = pltpu.make_async_remote_copy(
            src_ref=txs_ref,
            dst_ref=rxs_ref,
            send_sem=sc_sems.at[0],
            recv_sem=sc_sems.at[1],
            device_id=y_peer,
            device_id_type=pl.DeviceIdType.MESH,
        )
        sc.start()
        rdmas = []
        for c in range(C):
            r = pltpu.make_async_remote_copy(
                src_ref=txq_ref.at[pl.ds(c * CH, CH), :],
                dst_ref=rxq_ref.at[pl.ds(c * CH, CH), :],
                send_sem=send_sems.at[c],
                recv_sem=recv_sems.at[c],
                device_id=y_peer,
                device_id_type=pl.DeviceIdType.MESH,
            )
            r.start()
            rdmas.append(r)

        g = g_ref[:].reshape(1, D)
        cp_loc.wait()
        sc.wait_recv()
        rs = rxs_ref[:, :]

        stores = []
        for c in range(C):
            rdmas[c].wait_recv()
            ysum = (
                loc_ref[pl.ds(c * CH, CH), :]
                + rxq_ref[pl.ds(c * CH, CH), :].astype(jnp.float32) * rs
            )
            ms = jnp.mean(ysum * ysum, axis=-1, keepdims=True)
            ob_ref[pl.ds(c * CH, CH), :] = ysum * lax.rsqrt(ms + 1e-6) * g
            st = pltpu.make_async_copy(
                ob_ref.at[pl.ds(c * CH, CH), :],
                o_ref.at[pl.ds(c * CH, CH), :],
                st_sems.at[c],
            )
            st.start()
            stores.append(st)

        sc.wait_send()
        for c in range(C):
            rdmas[c].wait_send()
            stores[c].wait()

    return pl.pallas_call(
        body,
        out_shape=jax.ShapeDtypeStruct((M, D), jnp.float32),
        in_specs=[
            pl.BlockSpec(memory_space=pltpu.MemorySpace.HBM),
            pl.BlockSpec(memory_space=pltpu.VMEM),
        ],
        out_specs=pl.BlockSpec(memory_space=pltpu.MemorySpace.HBM),
        scratch_shapes=[
            pltpu.VMEM((M, D), jnp.float32),
            pltpu.VMEM((M, D), jnp.float32),
            pltpu.VMEM((M, D), jnp.float32),
            pltpu.VMEM((M, D), jnp.int8),
            pltpu.VMEM((M, D), jnp.int8),
            pltpu.VMEM((1, D), jnp.float32),
            pltpu.VMEM((1, D), jnp.float32),
            pltpu.SemaphoreType.DMA((2,)),
            pltpu.SemaphoreType.DMA((C,)),
            pltpu.SemaphoreType.DMA((2,)),
            pltpu.SemaphoreType.DMA((C,)),
            pltpu.SemaphoreType.DMA((C,)),
        ],
        compiler_params=pltpu.CompilerParams(collective_id=0),
    )(partial, gamma)
